# Initial kernel scaffold; baseline (speedup 1.0000x reference)
import jax
import jax.numpy as jnp
from jax.experimental import pallas as pl


def kernel(x_transaction, x_user, x_merchant, raw_txn_features, ei0_src, ei0_dst, ei1_src, ei1_dst, ei2_src, ei2_dst, ei3_src, ei3_dst, eps, params):
    raise NotImplementedError("write your pallas kernel here")


# speedup vs baseline: 1.0208x; 1.0208x over previous
import functools
import jax
import jax.numpy as jnp
from jax import lax
from jax.experimental import pallas as pl
from jax.experimental.pallas import tpu as pltpu

H, C = 4, 64
BM = 512


def _apply_act(x, act):
    if act == "none":
        return x
    if act == "relu":
        return jnp.maximum(x, 0.0)
    if act == "elu":
        return jnp.where(x > 0, x, jnp.exp(jnp.minimum(x, 0.0)) - 1.0)
    if act == "sigmoid":
        return jax.nn.sigmoid(x)
    raise ValueError(act)


def _mm_body(x_ref, w_ref, b_ref, o_ref, *, act):
    acc = jnp.dot(x_ref[...], w_ref[...], preferred_element_type=jnp.float32)
    o_ref[...] = _apply_act(acc + b_ref[...], act)


def _mm(x, w, b, act="none"):
    m, k = x.shape
    n = w.shape[1]
    mp = ((m + BM - 1) // BM) * BM
    np_ = ((n + 127) // 128) * 128
    if mp != m:
        x = jnp.pad(x, ((0, mp - m), (0, 0)))
    if np_ != n:
        w = jnp.pad(w, ((0, 0), (0, np_ - n)))
        b = jnp.pad(b, ((0, np_ - n),))
    out = pl.pallas_call(
        functools.partial(_mm_body, act=act),
        grid=(mp // BM,),
        in_specs=[
            pl.BlockSpec((BM, k), lambda i: (i, 0)),
            pl.BlockSpec((k, np_), lambda i: (0, 0)),
            pl.BlockSpec((1, np_), lambda i: (0, 0)),
        ],
        out_specs=pl.BlockSpec((BM, np_), lambda i: (i, 0)),
        out_shape=jax.ShapeDtypeStruct((mp, np_), jnp.float32),
    )(x, w, b.reshape(1, -1))
    return out[:m, :n]


def _ew2_body(a_ref, b_ref, c_ref, o_ref, *, act):
    o_ref[...] = _apply_act(a_ref[...] + b_ref[...] + c_ref[...], act)


def _ew_add2(a, b, bias, act="none"):
    m, n = a.shape
    mp = ((m + BM - 1) // BM) * BM
    if mp != m:
        a = jnp.pad(a, ((0, mp - m), (0, 0)))
        b = jnp.pad(b, ((0, mp - m), (0, 0)))
    out = pl.pallas_call(
        functools.partial(_ew2_body, act=act),
        grid=(mp // BM,),
        in_specs=[
            pl.BlockSpec((BM, n), lambda i: (i, 0)),
            pl.BlockSpec((BM, n), lambda i: (i, 0)),
            pl.BlockSpec((1, n), lambda i: (0, 0)),
        ],
        out_specs=pl.BlockSpec((BM, n), lambda i: (i, 0)),
        out_shape=jax.ShapeDtypeStruct((mp, n), jnp.float32),
    )(a, b, bias.reshape(1, -1))
    return out[:m]


def _z_body(mu_ref, lv_ref, eps_ref, o_ref):
    o_ref[...] = mu_ref[...] + jnp.exp(0.5 * lv_ref[...]) * eps_ref[...]


def _vae_z(mu, lv, eps):
    m, n = mu.shape
    mp = ((m + BM - 1) // BM) * BM
    pad = ((0, mp - m), (0, 0))
    out = pl.pallas_call(
        _z_body,
        grid=(mp // BM,),
        in_specs=[pl.BlockSpec((BM, n), lambda i: (i, 0))] * 3,
        out_specs=pl.BlockSpec((BM, n), lambda i: (i, 0)),
        out_shape=jax.ShapeDtypeStruct((mp, n), jnp.float32),
    )(jnp.pad(mu, pad), jnp.pad(lv, pad), jnp.pad(eps, pad))
    return out[:m]


def _sq_body(a_ref, b_ref, o_ref):
    d = a_ref[...] - b_ref[...]
    o_ref[...] = d * d


def _sqdiff(a, b):
    m, n = a.shape
    mp = ((m + BM - 1) // BM) * BM
    pad = ((0, mp - m), (0, 0))
    out = pl.pallas_call(
        _sq_body,
        grid=(mp // BM,),
        in_specs=[pl.BlockSpec((BM, n), lambda i: (i, 0))] * 2,
        out_specs=pl.BlockSpec((BM, n), lambda i: (i, 0)),
        out_shape=jax.ShapeDtypeStruct((mp, n), jnp.float32),
    )(jnp.pad(a, pad), jnp.pad(b, pad))
    return out[:m]


def _sig_body(a_ref, o_ref):
    o_ref[...] = jax.nn.sigmoid(a_ref[...])


def _sigmoid(a):
    m, n = a.shape
    mp = ((m + BM - 1) // BM) * BM
    out = pl.pallas_call(
        _sig_body,
        grid=(mp // BM,),
        in_specs=[pl.BlockSpec((BM, n), lambda i: (i, 0))],
        out_specs=pl.BlockSpec((BM, n), lambda i: (i, 0)),
        out_shape=jax.ShapeDtypeStruct((mp, n), jnp.float32),
    )(jnp.pad(a, ((0, mp - m), (0, 0))))
    return out[:m]


def _blockdiag(a):
    bd = jnp.zeros((H * C, H), jnp.float32)
    for h in range(H):
        bd = bd.at[h * C:(h + 1) * C, h].set(a[h])
    return bd


def _edge_phase(hs, asrc, adst, src, dst, n_dst):
    alpha = asrc[src] + adst[dst]
    alpha = jnp.where(alpha > 0, alpha, 0.2 * alpha)
    ex = jnp.exp(alpha)
    den = jax.ops.segment_sum(ex, dst, num_segments=n_dst)
    w = ex / (den[dst] + 1e-16)
    hsr = hs.reshape(-1, H, C)
    out = jax.ops.segment_sum(hsr[src] * w[:, :, None], dst, num_segments=n_dst)
    return out.reshape(n_dst, H * C)


_MEAN_M = None


def _mean_heads_mat():
    global _MEAN_M
    if _MEAN_M is None:
        m = jnp.zeros((H * C, C), jnp.float32)
        for h in range(H):
            m = m.at[h * C:(h + 1) * C, :].set(jnp.eye(C, dtype=jnp.float32) * 0.25)
        _MEAN_M = m
    return _MEAN_M


def kernel(x_transaction, x_user, x_merchant, raw_txn_features, ei0_src, ei0_dst, ei1_src, ei1_dst, ei2_src, ei2_dst, ei3_src, ei3_dst, eps, params):
    NN = {"transaction": x_transaction.shape[0], "user": x_user.shape[0], "merchant": x_merchant.shape[0]}
    ET = [("e0", "user", "transaction"), ("e1", "transaction", "user"), ("e2", "transaction", "merchant"), ("e3", "merchant", "transaction")]
    eidx = {"e0": (ei0_src, ei0_dst), "e1": (ei1_src, ei1_dst), "e2": (ei2_src, ei2_dst), "e3": (ei3_src, ei3_dst)}
    xd = {"transaction": x_transaction, "user": x_user, "merchant": x_merchant}
    gat = params["gat"]

    agg0 = {}
    for name, s, d in ET:
        p = gat["l0"][name]
        hs = _mm(xd[s], p["W_src"], jnp.zeros((H * C,), jnp.float32))
        hd = _mm(xd[d], p["W_dst"], jnp.zeros((H * C,), jnp.float32))
        asrc = _mm(hs, _blockdiag(p["a_src"]), jnp.zeros((H,), jnp.float32))
        adst = _mm(hd, _blockdiag(p["a_dst"]), jnp.zeros((H,), jnp.float32))
        o = _edge_phase(hs, asrc, adst, eidx[name][0], eidx[name][1], NN[d])
        agg0[d] = (o, p["b"]) if d not in agg0 else (agg0[d][0] + o, agg0[d][1] + p["b"])

    h0 = {}
    for d in ("transaction", "user", "merchant"):
        o, b = agg0[d]
        h0[d] = _ew_add2(o, jnp.zeros_like(o), b, act="elu")

    outs1 = []
    bias1 = jnp.zeros((C,), jnp.float32)
    for name, s, d in (ET[0], ET[3]):
        p = gat["l1"][name]
        hs = _mm(h0[s], p["W_src"], jnp.zeros((H * C,), jnp.float32))
        hd = _mm(h0[d], p["W_dst"], jnp.zeros((H * C,), jnp.float32))
        asrc = _mm(hs, _blockdiag(p["a_src"]), jnp.zeros((H,), jnp.float32))
        adst = _mm(hd, _blockdiag(p["a_dst"]), jnp.zeros((H,), jnp.float32))
        o = _edge_phase(hs, asrc, adst, eidx[name][0], eidx[name][1], NN[d])
        outs1.append(o)
        bias1 = bias1 + p["b"]

    summed1 = _ew_add2(outs1[0], outs1[1], jnp.zeros((H * C,), jnp.float32))
    h_t = _mm(summed1, _mean_heads_mat(), bias1)

    v = params["vae"]
    he = _mm(raw_txn_features, v["We1"], v["be1"], act="relu")
    mu = _mm(he, v["Wmu"], v["bmu"])
    logvar = _mm(he, v["Wlv"], v["blv"])
    z = _vae_z(mu, logvar, eps)
    hdec = _mm(z, v["Wd1"], v["bd1"], act="relu")
    x_recon = _mm(hdec, v["Wd2"], v["bd2"])

    sq = _sqdiff(raw_txn_features, x_recon)
    bn = params["bn"]
    sscale = bn["gamma"][0] / jnp.sqrt(bn["rv"][0] + 1e-5)
    wm = jnp.zeros((64, 128), jnp.float32)
    wm = wm.at[:, 0].set(1.0 / 64.0)
    wm = wm.at[:, 1].set(sscale / 64.0)
    bm = jnp.zeros((128,), jnp.float32)
    bm = bm.at[1].set(bn["beta"][0] - bn["rm"][0] * sscale)
    rcols = _mm(sq, wm, bm)
    recon_err = rcols[:, 0:1]
    recon_norm = rcols[:, 1:2]

    c = params["cls"]
    ci = jnp.concatenate([h_t, recon_norm], axis=1)
    w1p = jnp.pad(c["W1"], ((0, 63), (0, 0)))
    cip = jnp.pad(ci, ((0, 0), (0, 63)))
    hc = _mm(cip, w1p, c["b1"], act="elu")
    hc = _mm(hc, c["W2"], c["b2"], act="elu")
    logit2 = _mm(hc, c["W3"], c["b3"])
    logit = logit2[:, 0]
    fraud = _sigmoid(logit2[:, 0:1])[:, 0]
    return (logit, fraud, h_t, x_recon, mu, logvar, recon_err)


# speedup vs baseline: 5.3811x; 5.2717x over previous
import functools
import jax
import jax.numpy as jnp
from jax import lax
from jax.experimental import pallas as pl
from jax.experimental.pallas import tpu as pltpu
from jax.experimental.pallas import tpu_sc as plsc

H, C = 4, 64
BM = 512
KE = 512
NW = 32


def _pad128(n):
    return ((n + 1 + 127) // 128) * 128


def _sc_softmax_den(asrc16, adst16, src_p, dst_p, n_dst_pad, n_iters):
    e_pad = src_p.shape[0]
    rpt = n_dst_pad // 16
    mesh = plsc.VectorSubcoreMesh(core_axis_name="c", subcore_axis_name="s")

    @functools.partial(
        pl.kernel, mesh=mesh,
        compiler_params=pltpu.CompilerParams(use_tc_tiling_on_sc=False),
        out_type=[
            jax.ShapeDtypeStruct((e_pad, 16), jnp.float32),
            jax.ShapeDtypeStruct((2, n_dst_pad, 16), jnp.float32),
        ],
        scratch_types=[
            pltpu.VMEM((KE,), jnp.int32),
            pltpu.VMEM((KE,), jnp.int32),
            pltpu.VMEM((KE, 16), jnp.float32),
            pltpu.VMEM((KE, 16), jnp.float32),
            pltpu.VMEM((KE, 16), jnp.float32),
            pltpu.VMEM_SHARED((n_dst_pad, 16), jnp.float32),
            pltpu.SemaphoreType.DMA,
        ],
    )
    def k(asrc_hbm, adst_hbm, src_hbm, dst_hbm, zero_hbm, ex_hbm, den_hbm,
          src_v, dst_v, as_v, ad_v, ex_v, den_sh, sem):
        cid = lax.axis_index("c")
        sid = lax.axis_index("s")
        wid = sid * 2 + cid
        pltpu.sync_copy(zero_hbm.at[pl.ds(sid * rpt, rpt)],
                        den_sh.at[pl.ds(sid * rpt, rpt)])
        plsc.subcore_barrier()

        def step(it, _):
            base = it * (NW * KE) + wid * KE
            pltpu.sync_copy(src_hbm.at[pl.ds(base, KE)], src_v)
            pltpu.sync_copy(dst_hbm.at[pl.ds(base, KE)], dst_v)
            pltpu.async_copy(asrc_hbm.at[src_v], as_v, sem).wait()
            pltpu.async_copy(adst_hbm.at[dst_v], ad_v, sem).wait()

            def inner(v, _):
                al = as_v[v] + ad_v[v]
                al = jnp.where(al > 0, al, 0.2 * al)
                ex_v[v] = jnp.exp(al)
                return 0

            lax.fori_loop(0, KE, inner, 0)
            pltpu.sync_copy(ex_v, den_sh.at[dst_v], add=True)
            pltpu.sync_copy(ex_v, ex_hbm.at[pl.ds(base, KE)])
            return 0

        lax.fori_loop(0, n_iters, step, 0)
        plsc.subcore_barrier()
        pltpu.sync_copy(den_sh.at[pl.ds(sid * rpt, rpt)],
                        den_hbm.at[cid, pl.ds(sid * rpt, rpt)])

    zero16 = jnp.zeros((n_dst_pad, 16), jnp.float32)
    return k(asrc16, adst16, src_p, dst_p, zero16)


def _apply_act(x, act):
    if act == "none":
        return x
    if act == "relu":
        return jnp.maximum(x, 0.0)
    if act == "elu":
        return jnp.where(x > 0, x, jnp.exp(jnp.minimum(x, 0.0)) - 1.0)
    if act == "sigmoid":
        return jax.nn.sigmoid(x)
    raise ValueError(act)


def _mm_body(x_ref, w_ref, b_ref, o_ref, *, act):
    acc = jnp.dot(x_ref[...], w_ref[...], preferred_element_type=jnp.float32)
    o_ref[...] = _apply_act(acc + b_ref[...], act)


def _mm(x, w, b, act="none"):
    m, k = x.shape
    n = w.shape[1]
    mp = ((m + BM - 1) // BM) * BM
    np_ = ((n + 127) // 128) * 128
    if mp != m:
        x = jnp.pad(x, ((0, mp - m), (0, 0)))
    if np_ != n:
        w = jnp.pad(w, ((0, 0), (0, np_ - n)))
        b = jnp.pad(b, ((0, np_ - n),))
    out = pl.pallas_call(
        functools.partial(_mm_body, act=act),
        grid=(mp // BM,),
        in_specs=[
            pl.BlockSpec((BM, k), lambda i: (i, 0)),
            pl.BlockSpec((k, np_), lambda i: (0, 0)),
            pl.BlockSpec((1, np_), lambda i: (0, 0)),
        ],
        out_specs=pl.BlockSpec((BM, np_), lambda i: (i, 0)),
        out_shape=jax.ShapeDtypeStruct((mp, np_), jnp.float32),
    )(x, w, b.reshape(1, -1))
    return out[:m, :n]


def _ew2_body(a_ref, b_ref, c_ref, o_ref, *, act):
    o_ref[...] = _apply_act(a_ref[...] + b_ref[...] + c_ref[...], act)


def _ew_add2(a, b, bias, act="none"):
    m, n = a.shape
    mp = ((m + BM - 1) // BM) * BM
    if mp != m:
        a = jnp.pad(a, ((0, mp - m), (0, 0)))
        b = jnp.pad(b, ((0, mp - m), (0, 0)))
    out = pl.pallas_call(
        functools.partial(_ew2_body, act=act),
        grid=(mp // BM,),
        in_specs=[
            pl.BlockSpec((BM, n), lambda i: (i, 0)),
            pl.BlockSpec((BM, n), lambda i: (i, 0)),
            pl.BlockSpec((1, n), lambda i: (0, 0)),
        ],
        out_specs=pl.BlockSpec((BM, n), lambda i: (i, 0)),
        out_shape=jax.ShapeDtypeStruct((mp, n), jnp.float32),
    )(a, b, bias.reshape(1, -1))
    return out[:m]


def _z_body(mu_ref, lv_ref, eps_ref, o_ref):
    o_ref[...] = mu_ref[...] + jnp.exp(0.5 * lv_ref[...]) * eps_ref[...]


def _vae_z(mu, lv, eps):
    m, n = mu.shape
    mp = ((m + BM - 1) // BM) * BM
    pad = ((0, mp - m), (0, 0))
    out = pl.pallas_call(
        _z_body,
        grid=(mp // BM,),
        in_specs=[pl.BlockSpec((BM, n), lambda i: (i, 0))] * 3,
        out_specs=pl.BlockSpec((BM, n), lambda i: (i, 0)),
        out_shape=jax.ShapeDtypeStruct((mp, n), jnp.float32),
    )(jnp.pad(mu, pad), jnp.pad(lv, pad), jnp.pad(eps, pad))
    return out[:m]


def _sq_body(a_ref, b_ref, o_ref):
    d = a_ref[...] - b_ref[...]
    o_ref[...] = d * d


def _sqdiff(a, b):
    m, n = a.shape
    mp = ((m + BM - 1) // BM) * BM
    pad = ((0, mp - m), (0, 0))
    out = pl.pallas_call(
        _sq_body,
        grid=(mp // BM,),
        in_specs=[pl.BlockSpec((BM, n), lambda i: (i, 0))] * 2,
        out_specs=pl.BlockSpec((BM, n), lambda i: (i, 0)),
        out_shape=jax.ShapeDtypeStruct((mp, n), jnp.float32),
    )(jnp.pad(a, pad), jnp.pad(b, pad))
    return out[:m]


def _sig_body(a_ref, o_ref):
    o_ref[...] = jax.nn.sigmoid(a_ref[...])


def _sigmoid(a):
    m, n = a.shape
    mp = ((m + BM - 1) // BM) * BM
    out = pl.pallas_call(
        _sig_body,
        grid=(mp // BM,),
        in_specs=[pl.BlockSpec((BM, n), lambda i: (i, 0))],
        out_specs=pl.BlockSpec((BM, n), lambda i: (i, 0)),
        out_shape=jax.ShapeDtypeStruct((mp, n), jnp.float32),
    )(jnp.pad(a, ((0, mp - m), (0, 0))))
    return out[:m]


def _blockdiag(a):
    bd = jnp.zeros((H * C, H), jnp.float32)
    for h in range(H):
        bd = bd.at[h * C:(h + 1) * C, h].set(a[h])
    return bd


def _edge_phase(hs, asrc, adst, src, dst, n_dst):
    e = src.shape[0]
    n_src = asrc.shape[0]
    chunk = NW * KE
    e_pad = ((e + chunk - 1) // chunk) * chunk
    n_iters = e_pad // chunk
    n_src_pad = _pad128(n_src)
    n_dst_pad = _pad128(n_dst)
    asrc16 = jnp.pad(asrc, ((0, n_src_pad - n_src), (0, 12)))
    adst16 = jnp.pad(adst, ((0, n_dst_pad - n_dst), (0, 12)))
    src_p = jnp.concatenate([src, jnp.zeros((e_pad - e,), jnp.int32)])
    dst_p = jnp.concatenate([dst, jnp.full((e_pad - e,), n_dst, jnp.int32)])
    ex16, den2 = _sc_softmax_den(asrc16, adst16, src_p, dst_p, n_dst_pad, n_iters)
    den16 = _ew_add2(den2[0], den2[1], jnp.zeros((16,), jnp.float32))
    dgath = _sc_gather_rows(den16, dst_p, 16, n_iters)
    w16 = _ew_div(ex16, dgath)
    wexp = _mm(w16, _rep_mat(), jnp.zeros((H * C,), jnp.float32))
    n_iters_b = e_pad // (NW * KB)
    msg_raw = _sc_gather_rows(hs, src_p, 256, n_iters_b)
    msg = _ew_mul(msg_raw, wexp)
    n_pass = (n_dst + 2 * CH - 1) // (2 * CH)
    out = _sc_scatter_add(msg, dst_p, n_dst, n_iters_b, n_pass)
    out = out.reshape(n_pass * 2, CHP, 256)[:, :CH].reshape(-1, 256)
    return out[:n_dst]


KB = 256
CH = 3952
CHP = CH + 16


def _sc_gather_rows(table, idx, width, n_iters):
    e_pad = idx.shape[0]
    kpt = e_pad // (NW * n_iters)
    mesh = plsc.VectorSubcoreMesh(core_axis_name="c", subcore_axis_name="s")

    @functools.partial(
        pl.kernel, mesh=mesh,
        compiler_params=pltpu.CompilerParams(use_tc_tiling_on_sc=False),
        out_type=jax.ShapeDtypeStruct((e_pad, width), jnp.float32),
        scratch_types=[
            pltpu.VMEM((kpt,), jnp.int32),
            pltpu.VMEM((kpt, width), jnp.float32),
            pltpu.SemaphoreType.DMA,
        ],
    )
    def k(tab_hbm, idx_hbm, out_hbm, idx_v, rows_v, sem):
        cid = lax.axis_index("c")
        sid = lax.axis_index("s")
        wid = sid * 2 + cid

        def step(it, _):
            base = it * (NW * kpt) + wid * kpt
            pltpu.sync_copy(idx_hbm.at[pl.ds(base, kpt)], idx_v)
            pltpu.async_copy(tab_hbm.at[idx_v], rows_v, sem).wait()
            pltpu.sync_copy(rows_v, out_hbm.at[pl.ds(base, kpt)])
            return 0

        lax.fori_loop(0, n_iters, step, 0)

    return k(table, idx)


def _sc_scatter_add(msg, dst_p, n_dst, n_iters, n_pass):
    e_pad = dst_p.shape[0]
    mesh = plsc.VectorSubcoreMesh(core_axis_name="c", subcore_axis_name="s")
    zrpt = CHP // 16

    @functools.partial(
        pl.kernel, mesh=mesh,
        compiler_params=pltpu.CompilerParams(use_tc_tiling_on_sc=False),
        out_type=jax.ShapeDtypeStruct((n_pass * 2 * CHP, 256), jnp.float32),
        scratch_types=[
            pltpu.VMEM((KB,), jnp.int32),
            pltpu.VMEM((KB,), jnp.int32),
            pltpu.VMEM((KB, 256), jnp.float32),
            pltpu.VMEM_SHARED((CHP, 256), jnp.float32),
            pltpu.SemaphoreType.DMA,
        ],
    )
    def k(msg_hbm, dst_hbm, zero_hbm, out_hbm,
          dst_v, ldst_v, rows_v, acc_sh, sem):
        cid = lax.axis_index("c")
        sid = lax.axis_index("s")
        n_it = e_pad // (16 * KB)

        for p in range(n_pass):
            for c in range(2):
                base_row = p * (2 * CH) + c * CH

                @pl.when(cid == c)
                def _():
                    pltpu.sync_copy(zero_hbm.at[pl.ds(sid * zrpt, zrpt)],
                                    acc_sh.at[pl.ds(sid * zrpt, zrpt)])
            plsc.subcore_barrier()

            for c in range(2):
                base_row = p * (2 * CH) + c * CH

                @pl.when(cid == c)
                def _():
                    def step(it, _):
                        base = it * (16 * KB) + sid * KB
                        pltpu.sync_copy(dst_hbm.at[pl.ds(base, KB)], dst_v)
                        pltpu.sync_copy(msg_hbm.at[pl.ds(base, KB)], rows_v)

                        def lcalc(i, _):
                            d = dst_v[pl.ds(i * 16, 16)]
                            l = d - base_row
                            inr = (l >= 0) & (l < CH)
                            ldst_v[pl.ds(i * 16, 16)] = jnp.where(inr, l, CH)
                            return 0

                        lax.fori_loop(0, KB // 16, lcalc, 0)
                        pltpu.sync_copy(rows_v, acc_sh.at[ldst_v], add=True)
                        return 0

                    lax.fori_loop(0, n_it, step, 0)
            plsc.subcore_barrier()
            for c in range(2):
                out_base = (p * 2 + c) * CHP + sid * zrpt

                @pl.when(cid == c)
                def _():
                    pltpu.sync_copy(acc_sh.at[pl.ds(sid * zrpt, zrpt)],
                                    out_hbm.at[pl.ds(out_base, zrpt)])
            plsc.subcore_barrier()

    zero = jnp.zeros((CHP, 256), jnp.float32)
    return k(msg, dst_p, zero)


def _mul_body(a_ref, b_ref, o_ref):
    o_ref[...] = a_ref[...] * b_ref[...]


def _ew_mul(a, b):
    m, n = a.shape
    return pl.pallas_call(
        _mul_body,
        grid=(m // BM,),
        in_specs=[pl.BlockSpec((BM, n), lambda i: (i, 0))] * 2,
        out_specs=pl.BlockSpec((BM, n), lambda i: (i, 0)),
        out_shape=jax.ShapeDtypeStruct((m, n), jnp.float32),
    )(a, b)


def _div_body(a_ref, b_ref, o_ref):
    o_ref[...] = a_ref[...] / (b_ref[...] + 1e-16)


def _ew_div(a, b):
    m, n = a.shape
    return pl.pallas_call(
        _div_body,
        grid=(m // BM,),
        in_specs=[pl.BlockSpec((BM, n), lambda i: (i, 0))] * 2,
        out_specs=pl.BlockSpec((BM, n), lambda i: (i, 0)),
        out_shape=jax.ShapeDtypeStruct((m, n), jnp.float32),
    )(a, b)


_REP_M = None


def _rep_mat():
    global _REP_M
    if _REP_M is None:
        r = jnp.zeros((16, H * C), jnp.float32)
        for h in range(H):
            r = r.at[h, h * C:(h + 1) * C].set(1.0)
        _REP_M = r
    return _REP_M


_MEAN_M = None


def _mean_heads_mat():
    global _MEAN_M
    if _MEAN_M is None:
        m = jnp.zeros((H * C, C), jnp.float32)
        for h in range(H):
            m = m.at[h * C:(h + 1) * C, :].set(jnp.eye(C, dtype=jnp.float32) * 0.25)
        _MEAN_M = m
    return _MEAN_M


def kernel(x_transaction, x_user, x_merchant, raw_txn_features, ei0_src, ei0_dst, ei1_src, ei1_dst, ei2_src, ei2_dst, ei3_src, ei3_dst, eps, params):
    NN = {"transaction": x_transaction.shape[0], "user": x_user.shape[0], "merchant": x_merchant.shape[0]}
    ET = [("e0", "user", "transaction"), ("e1", "transaction", "user"), ("e2", "transaction", "merchant"), ("e3", "merchant", "transaction")]
    eidx = {"e0": (ei0_src, ei0_dst), "e1": (ei1_src, ei1_dst), "e2": (ei2_src, ei2_dst), "e3": (ei3_src, ei3_dst)}
    xd = {"transaction": x_transaction, "user": x_user, "merchant": x_merchant}
    gat = params["gat"]

    agg0 = {}
    for name, s, d in ET:
        p = gat["l0"][name]
        hs = _mm(xd[s], p["W_src"], jnp.zeros((H * C,), jnp.float32))
        hd = _mm(xd[d], p["W_dst"], jnp.zeros((H * C,), jnp.float32))
        asrc = _mm(hs, _blockdiag(p["a_src"]), jnp.zeros((H,), jnp.float32))
        adst = _mm(hd, _blockdiag(p["a_dst"]), jnp.zeros((H,), jnp.float32))
        o = _edge_phase(hs, asrc, adst, eidx[name][0], eidx[name][1], NN[d])
        agg0[d] = (o, p["b"]) if d not in agg0 else (agg0[d][0] + o, agg0[d][1] + p["b"])

    h0 = {}
    for d in ("transaction", "user", "merchant"):
        o, b = agg0[d]
        h0[d] = _ew_add2(o, jnp.zeros_like(o), b, act="elu")

    outs1 = []
    bias1 = jnp.zeros((C,), jnp.float32)
    for name, s, d in (ET[0], ET[3]):
        p = gat["l1"][name]
        hs = _mm(h0[s], p["W_src"], jnp.zeros((H * C,), jnp.float32))
        hd = _mm(h0[d], p["W_dst"], jnp.zeros((H * C,), jnp.float32))
        asrc = _mm(hs, _blockdiag(p["a_src"]), jnp.zeros((H,), jnp.float32))
        adst = _mm(hd, _blockdiag(p["a_dst"]), jnp.zeros((H,), jnp.float32))
        o = _edge_phase(hs, asrc, adst, eidx[name][0], eidx[name][1], NN[d])
        outs1.append(o)
        bias1 = bias1 + p["b"]

    summed1 = _ew_add2(outs1[0], outs1[1], jnp.zeros((H * C,), jnp.float32))
    h_t = _mm(summed1, _mean_heads_mat(), bias1)

    v = params["vae"]
    he = _mm(raw_txn_features, v["We1"], v["be1"], act="relu")
    mu = _mm(he, v["Wmu"], v["bmu"])
    logvar = _mm(he, v["Wlv"], v["blv"])
    z = _vae_z(mu, logvar, eps)
    hdec = _mm(z, v["Wd1"], v["bd1"], act="relu")
    x_recon = _mm(hdec, v["Wd2"], v["bd2"])

    sq = _sqdiff(raw_txn_features, x_recon)
    bn = params["bn"]
    sscale = bn["gamma"][0] / jnp.sqrt(bn["rv"][0] + 1e-5)
    wm = jnp.zeros((64, 128), jnp.float32)
    wm = wm.at[:, 0].set(1.0 / 64.0)
    wm = wm.at[:, 1].set(sscale / 64.0)
    bm = jnp.zeros((128,), jnp.float32)
    bm = bm.at[1].set(bn["beta"][0] - bn["rm"][0] * sscale)
    rcols = _mm(sq, wm, bm)
    recon_err = rcols[:, 0:1]
    recon_norm = rcols[:, 1:2]

    c = params["cls"]
    ci = jnp.concatenate([h_t, recon_norm], axis=1)
    w1p = jnp.pad(c["W1"], ((0, 63), (0, 0)))
    cip = jnp.pad(ci, ((0, 0), (0, 63)))
    hc = _mm(cip, w1p, c["b1"], act="elu")
    hc = _mm(hc, c["W2"], c["b2"], act="elu")
    logit2 = _mm(hc, c["W3"], c["b3"])
    logit = logit2[:, 0]
    fraud = _sigmoid(logit2[:, 0:1])[:, 0]
    return (logit, fraud, h_t, x_recon, mu, logvar, recon_err)
